# baseline (speedup 1.0000x reference)
import functools

import jax
import jax.numpy as jnp
from jax import lax
from jax.experimental import pallas as pl
from jax.experimental.pallas import tpu as pltpu
from jax.experimental.pallas import tpu_sc as plsc

H = 64
EPS = 1e-5
BN = 1024
G = 128
S = 1024
NG = 2048


def _rup(n, m):
    return ((n + m - 1) // m) * m


@functools.lru_cache(maxsize=None)
def _mk_segsum(n_x, e_pad, m_pad, nblk, br):
    assert e_pad % (16 * S) == 0 and m_pad == 2 * nblk * br and br % 16 == 0
    et = e_pad // 16
    nslab = et // S
    nsub = S // G
    brt = (br + G) // 16
    brs = br // 16
    mesh = plsc.VectorSubcoreMesh(core_axis_name="c", subcore_axis_name="s")

    @functools.partial(
        pl.kernel, mesh=mesh,
        compiler_params=pltpu.CompilerParams(use_tc_tiling_on_sc=False),
        out_type=jax.ShapeDtypeStruct((m_pad, H), jnp.float32),
        scratch_types=[
            pltpu.VMEM((S,), jnp.int32),
            pltpu.VMEM((S,), jnp.int32),
            pltpu.VMEM((G,), jnp.int32),
            pltpu.VMEM((G,), jnp.int32),
            pltpu.VMEM((G, H), jnp.float32),
            pltpu.VMEM((G, H), jnp.float32),
            pltpu.VMEM((G, H), jnp.float32),
            pltpu.VMEM_SHARED((br + G, H), jnp.float32),
            pltpu.SemaphoreType.DMA,
            pltpu.SemaphoreType.DMA,
        ],
    )
    def seg(x_hbm, src_hbm, dst_hbm, out_hbm,
            dslab, sslab, dstm0, dstm1, rows0, rows1, zbuf, acc, sem0, sem1):
        cid = lax.axis_index("c")
        sid = lax.axis_index("s")
        dstm = [dstm0, dstm1]
        rows = [rows0, rows1]
        sem = [sem0, sem1]
        zv = jnp.zeros((16,), jnp.float32)

        def zrow(r, _):
            for v in range(H // 16):
                zbuf[r, pl.ds(v * 16, 16)] = zv
            return 0
        lax.fori_loop(0, G, zrow, 0)

        e0 = sid * et
        for blk in range(nblk):
            lo = (cid * nblk + blk) * br
            nz = brt // G
            rem = brt - nz * G

            def zb(z, _):
                pltpu.sync_copy(zbuf, acc.at[pl.ds(sid * brt + z * G, G)])
                return 0
            lax.fori_loop(0, nz, zb, 0)
            if rem:
                pltpu.sync_copy(zbuf.at[pl.ds(0, rem)],
                                acc.at[pl.ds(sid * brt + nz * G, rem)])
            plsc.subcore_barrier()

            def slab_body(t, _):
                eoff = e0 + t * S
                pltpu.sync_copy(dst_hbm.at[pl.ds(eoff, S)], dslab)
                pltpu.sync_copy(src_hbm.at[pl.ds(eoff, S)], sslab)
                cps = []
                for k in range(nsub):
                    p = k % 2
                    for v in range(G // 16):
                        dv = dslab[pl.ds(k * G + v * 16, 16)]
                        inr = (dv >= lo) & (dv < lo + br)
                        dstm[p][pl.ds(v * 16, 16)] = jnp.where(inr, dv - lo, br)
                    cps.append(pltpu.async_copy(
                        x_hbm.at[sslab.at[pl.ds(k * G, G)]], rows[p], sem[p]))
                    if k > 0:
                        cps[k - 1].wait()
                        pltpu.sync_copy(rows[1 - p], acc.at[dstm[1 - p]],
                                        add=True)
                cps[nsub - 1].wait()
                q = (nsub - 1) % 2
                pltpu.sync_copy(rows[q], acc.at[dstm[q]], add=True)
                return 0
            lax.fori_loop(0, nslab, slab_body, 0)
            plsc.subcore_barrier()
            pltpu.sync_copy(acc.at[pl.ds(sid * brs, brs)],
                            out_hbm.at[pl.ds(lo + sid * brs, brs)])
            plsc.subcore_barrier()

    return seg


def _stats_update(i, y, s_ref, nreal):
    rows = pl.program_id(0) * BN + lax.broadcasted_iota(jnp.int32, (BN, 1), 0)
    ym = jnp.where(rows < nreal, y, 0.0)
    s = jnp.stack([ym.sum(0), (ym * ym).sum(0)], axis=0)

    @pl.when(i == 0)
    def _():
        s_ref[...] = s

    @pl.when(i > 0)
    def _():
        s_ref[...] += s


def _pack_mm(x, aggu, aggb, w, b, nreal):
    np_, _ = x.shape
    grid = np_ // BN
    has_u, has_b = aggu is not None, aggb is not None

    def body(*refs):
        i = pl.program_id(0)
        x_ = refs[0][...]
        k = 1
        if has_u:
            xu = x_ + refs[k][...]
            k += 1
        else:
            xu = x_
        if has_b:
            xb = x_ + refs[k][...]
            k += 1
        else:
            xb = x_
        w_, b_ = refs[k][...], refs[k + 1][...]
        y_ref, s_ref = refs[k + 2], refs[k + 3]
        cat = jnp.concatenate([xu, xb], axis=1)
        y = jnp.dot(cat, w_, preferred_element_type=jnp.float32) + b_
        y_ref[...] = y
        _stats_update(i, y, s_ref, nreal)

    ins = [x] + ([aggu] if has_u else []) + ([aggb] if has_b else []) + [w, b]
    in_specs = ([pl.BlockSpec((BN, H), lambda i: (i, 0))] * (1 + has_u + has_b)
                + [pl.BlockSpec((2 * H, 2 * H), lambda i: (0, 0)),
                   pl.BlockSpec((1, 2 * H), lambda i: (0, 0))])
    return pl.pallas_call(
        body, grid=(grid,),
        in_specs=in_specs,
        out_specs=[pl.BlockSpec((BN, 2 * H), lambda i: (i, 0)),
                   pl.BlockSpec((2, 2 * H), lambda i: (0, 0))],
        out_shape=[jax.ShapeDtypeStruct((np_, 2 * H), jnp.float32),
                   jax.ShapeDtypeStruct((2, 2 * H), jnp.float32)],
    )(*ins)


def _bn_scale_shift(s, g, be, nreal):
    mean = s[0:1] / nreal
    var = s[1:2] / nreal - mean * mean
    inv = g * lax.rsqrt(var + EPS)
    return inv, be - mean * inv


def _bn_mm(y_in, s_in, g, be, w, b, nreal):
    np_, hin = y_in.shape
    hout = w.shape[1]
    grid = np_ // BN

    def body(y_ref, s_in_ref, g_ref, be_ref, w_ref, b_ref, o_ref, s_ref):
        i = pl.program_id(0)
        inv, sh = _bn_scale_shift(s_in_ref[...], g_ref[...], be_ref[...], nreal)
        h = jnp.maximum(y_ref[...] * inv + sh, 0.0)
        o = jnp.dot(h, w_ref[...], preferred_element_type=jnp.float32) + b_ref[...]
        o_ref[...] = o
        _stats_update(i, o, s_ref, nreal)

    return pl.pallas_call(
        body, grid=(grid,),
        in_specs=[pl.BlockSpec((BN, hin), lambda i: (i, 0)),
                  pl.BlockSpec((2, hin), lambda i: (0, 0)),
                  pl.BlockSpec((1, hin), lambda i: (0, 0)),
                  pl.BlockSpec((1, hin), lambda i: (0, 0)),
                  pl.BlockSpec((hin, hout), lambda i: (0, 0)),
                  pl.BlockSpec((1, hout), lambda i: (0, 0))],
        out_specs=[pl.BlockSpec((BN, hout), lambda i: (i, 0)),
                   pl.BlockSpec((2, hout), lambda i: (0, 0))],
        out_shape=[jax.ShapeDtypeStruct((np_, hout), jnp.float32),
                   jax.ShapeDtypeStruct((2, hout), jnp.float32)],
    )(y_in, s_in, g, be, w, b)


def _bn_act(y_in, s_in, g, be, nreal):
    np_, hin = y_in.shape
    grid = np_ // BN

    def body(y_ref, s_in_ref, g_ref, be_ref, o_ref):
        inv, sh = _bn_scale_shift(s_in_ref[...], g_ref[...], be_ref[...], nreal)
        o_ref[...] = jnp.maximum(y_ref[...] * inv + sh, 0.0)

    return pl.pallas_call(
        body, grid=(grid,),
        in_specs=[pl.BlockSpec((BN, hin), lambda i: (i, 0)),
                  pl.BlockSpec((2, hin), lambda i: (0, 0)),
                  pl.BlockSpec((1, hin), lambda i: (0, 0)),
                  pl.BlockSpec((1, hin), lambda i: (0, 0))],
        out_specs=pl.BlockSpec((BN, hin), lambda i: (i, 0)),
        out_shape=jax.ShapeDtypeStruct((np_, hin), jnp.float32),
    )(y_in, s_in, g, be)


def _pool_lin(pools, ws, bs):
    def body(p0, p1, p2, w0, w1, w2, b0, b1, b2, o_ref):
        o = jnp.maximum(jnp.dot(p0[...], w0[...],
                                preferred_element_type=jnp.float32) + b0[...], 0.0)
        o += jnp.maximum(jnp.dot(p1[...], w1[...],
                                 preferred_element_type=jnp.float32) + b1[...], 0.0)
        o += jnp.maximum(jnp.dot(p2[...], w2[...],
                                 preferred_element_type=jnp.float32) + b2[...], 0.0)
        o_ref[...] = o

    specs = ([pl.BlockSpec((NG, H), lambda: (0, 0))] * 3
             + [pl.BlockSpec((H, 2 * H), lambda: (0, 0))] * 3
             + [pl.BlockSpec((1, 2 * H), lambda: (0, 0))] * 3)
    return pl.pallas_call(
        body,
        in_specs=specs,
        out_specs=pl.BlockSpec((NG, 2 * H), lambda: (0, 0)),
        out_shape=jax.ShapeDtypeStruct((NG, 2 * H), jnp.float32),
    )(*pools, *ws, *bs)


def _blockdiag(a, b):
    z = jnp.zeros_like(a)
    return jnp.concatenate(
        [jnp.concatenate([a, z], 1), jnp.concatenate([z, b], 1)], 0)


def _pad_edges(src, dst, e_pad):
    e = src.shape[0]
    ps = jnp.concatenate([src.astype(jnp.int32),
                          jnp.zeros((e_pad - e,), jnp.int32)])
    pd = jnp.concatenate([dst.astype(jnp.int32),
                          jnp.full((e_pad - e,), -1, jnp.int32)])
    return ps, pd


def kernel(params, v_x, up_index_0, boundary_index_1, up_index_1,
           boundary_index_2, batch0, batch1, batch2):
    V, E1, C2 = v_x.shape[0], batch1.shape[0], batch2.shape[0]
    Vp, E1p, C2p = _rup(V, BN), _rup(E1, BN), _rup(C2, BN)

    seg_v = _mk_segsum(params['v_embed'].shape[0], _rup(V, 16 * S), Vp, 1, Vp // 2)
    seg_b1 = _mk_segsum(Vp, _rup(200000, 16 * S), E1p, 2, E1p // 4)
    seg_b2 = _mk_segsum(E1p, _rup(120000, 16 * S), C2p, 1, C2p // 2)
    seg_u0 = _mk_segsum(Vp, _rup(200000, 16 * S), Vp, 1, Vp // 2)
    seg_u1 = _mk_segsum(E1p, _rup(400000, 16 * S), E1p, 2, E1p // 4)
    seg_p0 = _mk_segsum(Vp, _rup(V, 16 * S), NG, 1, NG // 2)
    seg_p1 = _mk_segsum(E1p, _rup(E1, 16 * S), NG, 1, NG // 2)
    seg_p2 = _mk_segsum(C2p, _rup(C2, 16 * S), NG, 1, NG // 2)

    s_em, d_em = _pad_edges(v_x, jnp.arange(V, dtype=jnp.int32), _rup(V, 16 * S))
    s_b1, d_b1 = _pad_edges(boundary_index_1[0], boundary_index_1[1],
                            _rup(200000, 16 * S))
    s_b2, d_b2 = _pad_edges(boundary_index_2[0], boundary_index_2[1],
                            _rup(120000, 16 * S))
    s_u0, d_u0 = _pad_edges(up_index_0[0], up_index_0[1], _rup(200000, 16 * S))
    s_u1, d_u1 = _pad_edges(up_index_1[0], up_index_1[1], _rup(400000, 16 * S))
    s_p0, d_p0 = _pad_edges(jnp.arange(V, dtype=jnp.int32), batch0, _rup(V, 16 * S))
    s_p1, d_p1 = _pad_edges(jnp.arange(E1, dtype=jnp.int32), batch1,
                            _rup(E1, 16 * S))
    s_p2, d_p2 = _pad_edges(jnp.arange(C2, dtype=jnp.int32), batch2,
                            _rup(C2, 16 * S))

    x0 = seg_v(params['v_embed'], s_em, d_em)
    x1 = seg_b1(x0, s_b1, d_b1)
    x2 = seg_b2(x1, s_b2, d_b2)

    def pk(p):
        u, bd = p['up'], p['bdry']
        return dict(
            W1=_blockdiag(u['W1'], bd['W1']),
            b1=jnp.concatenate([u['b1'], bd['b1']]).reshape(1, 2 * H),
            g1=jnp.concatenate([u['g1'], bd['g1']]).reshape(1, 2 * H),
            be1=jnp.concatenate([u['be1'], bd['be1']]).reshape(1, 2 * H),
            W2=_blockdiag(u['W2'], bd['W2']),
            b2=jnp.concatenate([u['b2'], bd['b2']]).reshape(1, 2 * H),
            g2=jnp.concatenate([u['g2'], bd['g2']]).reshape(1, 2 * H),
            be2=jnp.concatenate([u['be2'], bd['be2']]).reshape(1, 2 * H),
            Wc=p['comb']['W'], bc=p['comb']['b'].reshape(1, H),
            gc=p['comb']['g'].reshape(1, H), bec=p['comb']['be'].reshape(1, H),
        )

    xs = [x0, x1, x2]
    n_real = [V, E1, C2]
    for l in range(3):
        lp = params['layers'][l]
        aggu = [seg_u0(xs[0], s_u0, d_u0), seg_u1(xs[1], s_u1, d_u1), None]
        aggb = [None, seg_b1(xs[0], s_b1, d_b1), seg_b2(xs[1], s_b2, d_b2)]
        new_xs = []
        for d in range(3):
            p = pk(lp[d])
            n = n_real[d]
            y1, s1 = _pack_mm(xs[d], aggu[d], aggb[d], p['W1'], p['b1'], n)
            y2, s2 = _bn_mm(y1, s1, p['g1'], p['be1'], p['W2'], p['b2'], n)
            y3, s3 = _bn_mm(y2, s2, p['g2'], p['be2'], p['Wc'], p['bc'], n)
            new_xs.append(_bn_act(y3, s3, p['gc'], p['bec'], n))
        xs = new_xs

    pools = [seg_p0(xs[0], s_p0, d_p0), seg_p1(xs[1], s_p1, d_p1),
             seg_p2(xs[2], s_p2, d_p2)]
    ws = [w for w in params['lin1_W']]
    bs = [b.reshape(1, 2 * H) for b in params['lin1_b']]
    return _pool_lin(pools, ws, bs)


# speedup vs baseline: 1.0003x; 1.0003x over previous
import functools

import jax
import jax.numpy as jnp
from jax import lax
from jax.experimental import pallas as pl
from jax.experimental.pallas import tpu as pltpu
from jax.experimental.pallas import tpu_sc as plsc

H = 64
EPS = 1e-5
BN = 1024
G = 128
S = 1024
NG = 2048


def _rup(n, m):
    return ((n + m - 1) // m) * m


@functools.lru_cache(maxsize=None)
def _mk_segsum(n_x, e_pad, m_pad, nblk, br):
    assert e_pad % (16 * S) == 0 and m_pad == 2 * nblk * br and br % 16 == 0
    et = e_pad // 16
    nslab = et // S
    nsub = S // G
    brt = (br + G) // 16
    brs = br // 16
    mesh = plsc.VectorSubcoreMesh(core_axis_name="c", subcore_axis_name="s")

    @functools.partial(
        pl.kernel, mesh=mesh,
        compiler_params=pltpu.CompilerParams(use_tc_tiling_on_sc=False),
        out_type=jax.ShapeDtypeStruct((m_pad, H), jnp.float32),
        scratch_types=[
            pltpu.VMEM((S,), jnp.int32),
            pltpu.VMEM((S,), jnp.int32),
            pltpu.VMEM((G,), jnp.int32),
            pltpu.VMEM((G,), jnp.int32),
            pltpu.VMEM((G,), jnp.int32),
            pltpu.VMEM((G,), jnp.int32),
            pltpu.VMEM((G, H), jnp.float32),
            pltpu.VMEM((G, H), jnp.float32),
            pltpu.VMEM((G, H), jnp.float32),
            pltpu.VMEM_SHARED((br + G, H), jnp.float32),
            pltpu.SemaphoreType.DMA,
            pltpu.SemaphoreType.DMA,
        ],
    )
    def seg(x_hbm, src_hbm, dst_hbm, out_hbm,
            dslab, sslab, dstm0, dstm1, gidx0, gidx1, rows0, rows1,
            zbuf, acc, sem0, sem1):
        cid = lax.axis_index("c")
        sid = lax.axis_index("s")
        dstm = [dstm0, dstm1]
        gidx = [gidx0, gidx1]
        rows = [rows0, rows1]
        sem = [sem0, sem1]
        zv = jnp.zeros((16,), jnp.float32)

        def zrow(r, _):
            for v in range(H // 16):
                zbuf[r, pl.ds(v * 16, 16)] = zv
            return 0
        lax.fori_loop(0, G, zrow, 0)

        e0 = sid * et
        for blk in range(nblk):
            lo = (cid * nblk + blk) * br
            nz = brt // G
            rem = brt - nz * G

            def zb(z, _):
                pltpu.sync_copy(zbuf, acc.at[pl.ds(sid * brt + z * G, G)])
                return 0
            lax.fori_loop(0, nz, zb, 0)
            if rem:
                pltpu.sync_copy(zbuf.at[pl.ds(0, rem)],
                                acc.at[pl.ds(sid * brt + nz * G, rem)])
            plsc.subcore_barrier()

            def slab_body(t, _):
                eoff = e0 + t * S
                pltpu.sync_copy(dst_hbm.at[pl.ds(eoff, S)], dslab)
                pltpu.sync_copy(src_hbm.at[pl.ds(eoff, S)], sslab)
                cps = []
                for k in range(nsub):
                    p = k % 2
                    for v in range(G // 16):
                        dv = dslab[pl.ds(k * G + v * 16, 16)]
                        inr = (dv >= lo) & (dv < lo + br)
                        dstm[p][pl.ds(v * 16, 16)] = jnp.where(inr, dv - lo, br)
                        gidx[p][pl.ds(v * 16, 16)] = sslab[pl.ds(k * G + v * 16, 16)]
                    cps.append(pltpu.async_copy(
                        x_hbm.at[gidx[p]], rows[p], sem[p]))
                    if k > 0:
                        cps[k - 1].wait()
                        pltpu.sync_copy(rows[1 - p], acc.at[dstm[1 - p]],
                                        add=True)
                cps[nsub - 1].wait()
                q = (nsub - 1) % 2
                pltpu.sync_copy(rows[q], acc.at[dstm[q]], add=True)
                return 0
            lax.fori_loop(0, nslab, slab_body, 0)
            plsc.subcore_barrier()
            pltpu.sync_copy(acc.at[pl.ds(sid * brs, brs)],
                            out_hbm.at[pl.ds(lo + sid * brs, brs)])
            plsc.subcore_barrier()

    return seg


def _stats_update(i, y, s_ref, nreal):
    rows = pl.program_id(0) * BN + lax.broadcasted_iota(jnp.int32, (BN, 1), 0)
    ym = jnp.where(rows < nreal, y, 0.0)
    s = jnp.stack([ym.sum(0), (ym * ym).sum(0)], axis=0)

    @pl.when(i == 0)
    def _():
        s_ref[...] = s

    @pl.when(i > 0)
    def _():
        s_ref[...] += s


def _pack_mm(x, aggu, aggb, w, b, nreal):
    np_, _ = x.shape
    grid = np_ // BN
    has_u, has_b = aggu is not None, aggb is not None

    def body(*refs):
        i = pl.program_id(0)
        x_ = refs[0][...]
        k = 1
        if has_u:
            xu = x_ + refs[k][...]
            k += 1
        else:
            xu = x_
        if has_b:
            xb = x_ + refs[k][...]
            k += 1
        else:
            xb = x_
        w_, b_ = refs[k][...], refs[k + 1][...]
        y_ref, s_ref = refs[k + 2], refs[k + 3]
        cat = jnp.concatenate([xu, xb], axis=1)
        y = jnp.dot(cat, w_, preferred_element_type=jnp.float32) + b_
        y_ref[...] = y
        _stats_update(i, y, s_ref, nreal)

    ins = [x] + ([aggu] if has_u else []) + ([aggb] if has_b else []) + [w, b]
    in_specs = ([pl.BlockSpec((BN, H), lambda i: (i, 0))] * (1 + has_u + has_b)
                + [pl.BlockSpec((2 * H, 2 * H), lambda i: (0, 0)),
                   pl.BlockSpec((1, 2 * H), lambda i: (0, 0))])
    return pl.pallas_call(
        body, grid=(grid,),
        in_specs=in_specs,
        out_specs=[pl.BlockSpec((BN, 2 * H), lambda i: (i, 0)),
                   pl.BlockSpec((2, 2 * H), lambda i: (0, 0))],
        out_shape=[jax.ShapeDtypeStruct((np_, 2 * H), jnp.float32),
                   jax.ShapeDtypeStruct((2, 2 * H), jnp.float32)],
    )(*ins)


def _bn_scale_shift(s, g, be, nreal):
    mean = s[0:1] / nreal
    var = s[1:2] / nreal - mean * mean
    inv = g * lax.rsqrt(var + EPS)
    return inv, be - mean * inv


def _bn_mm(y_in, s_in, g, be, w, b, nreal):
    np_, hin = y_in.shape
    hout = w.shape[1]
    grid = np_ // BN

    def body(y_ref, s_in_ref, g_ref, be_ref, w_ref, b_ref, o_ref, s_ref):
        i = pl.program_id(0)
        inv, sh = _bn_scale_shift(s_in_ref[...], g_ref[...], be_ref[...], nreal)
        h = jnp.maximum(y_ref[...] * inv + sh, 0.0)
        o = jnp.dot(h, w_ref[...], preferred_element_type=jnp.float32) + b_ref[...]
        o_ref[...] = o
        _stats_update(i, o, s_ref, nreal)

    return pl.pallas_call(
        body, grid=(grid,),
        in_specs=[pl.BlockSpec((BN, hin), lambda i: (i, 0)),
                  pl.BlockSpec((2, hin), lambda i: (0, 0)),
                  pl.BlockSpec((1, hin), lambda i: (0, 0)),
                  pl.BlockSpec((1, hin), lambda i: (0, 0)),
                  pl.BlockSpec((hin, hout), lambda i: (0, 0)),
                  pl.BlockSpec((1, hout), lambda i: (0, 0))],
        out_specs=[pl.BlockSpec((BN, hout), lambda i: (i, 0)),
                   pl.BlockSpec((2, hout), lambda i: (0, 0))],
        out_shape=[jax.ShapeDtypeStruct((np_, hout), jnp.float32),
                   jax.ShapeDtypeStruct((2, hout), jnp.float32)],
    )(y_in, s_in, g, be, w, b)


def _bn_act(y_in, s_in, g, be, nreal):
    np_, hin = y_in.shape
    grid = np_ // BN

    def body(y_ref, s_in_ref, g_ref, be_ref, o_ref):
        inv, sh = _bn_scale_shift(s_in_ref[...], g_ref[...], be_ref[...], nreal)
        o_ref[...] = jnp.maximum(y_ref[...] * inv + sh, 0.0)

    return pl.pallas_call(
        body, grid=(grid,),
        in_specs=[pl.BlockSpec((BN, hin), lambda i: (i, 0)),
                  pl.BlockSpec((2, hin), lambda i: (0, 0)),
                  pl.BlockSpec((1, hin), lambda i: (0, 0)),
                  pl.BlockSpec((1, hin), lambda i: (0, 0))],
        out_specs=pl.BlockSpec((BN, hin), lambda i: (i, 0)),
        out_shape=jax.ShapeDtypeStruct((np_, hin), jnp.float32),
    )(y_in, s_in, g, be)


def _pool_lin(pools, ws, bs):
    def body(p0, p1, p2, w0, w1, w2, b0, b1, b2, o_ref):
        o = jnp.maximum(jnp.dot(p0[...], w0[...],
                                preferred_element_type=jnp.float32) + b0[...], 0.0)
        o += jnp.maximum(jnp.dot(p1[...], w1[...],
                                 preferred_element_type=jnp.float32) + b1[...], 0.0)
        o += jnp.maximum(jnp.dot(p2[...], w2[...],
                                 preferred_element_type=jnp.float32) + b2[...], 0.0)
        o_ref[...] = o

    specs = ([pl.BlockSpec((NG, H), lambda: (0, 0))] * 3
             + [pl.BlockSpec((H, 2 * H), lambda: (0, 0))] * 3
             + [pl.BlockSpec((1, 2 * H), lambda: (0, 0))] * 3)
    return pl.pallas_call(
        body,
        in_specs=specs,
        out_specs=pl.BlockSpec((NG, 2 * H), lambda: (0, 0)),
        out_shape=jax.ShapeDtypeStruct((NG, 2 * H), jnp.float32),
    )(*pools, *ws, *bs)


def _blockdiag(a, b):
    z = jnp.zeros_like(a)
    return jnp.concatenate(
        [jnp.concatenate([a, z], 1), jnp.concatenate([z, b], 1)], 0)


def _pad_edges(src, dst, e_pad):
    e = src.shape[0]
    ps = jnp.concatenate([src.astype(jnp.int32),
                          jnp.zeros((e_pad - e,), jnp.int32)])
    pd = jnp.concatenate([dst.astype(jnp.int32),
                          jnp.full((e_pad - e,), -1, jnp.int32)])
    return ps, pd


def kernel(params, v_x, up_index_0, boundary_index_1, up_index_1,
           boundary_index_2, batch0, batch1, batch2):
    V, E1, C2 = v_x.shape[0], batch1.shape[0], batch2.shape[0]
    Vp, E1p, C2p = _rup(V, BN), _rup(E1, BN), _rup(C2, BN)

    seg_v = _mk_segsum(params['v_embed'].shape[0], _rup(V, 16 * S), Vp, 1, Vp // 2)
    seg_b1 = _mk_segsum(Vp, _rup(200000, 16 * S), E1p, 2, E1p // 4)
    seg_b2 = _mk_segsum(E1p, _rup(120000, 16 * S), C2p, 1, C2p // 2)
    seg_u0 = _mk_segsum(Vp, _rup(200000, 16 * S), Vp, 1, Vp // 2)
    seg_u1 = _mk_segsum(E1p, _rup(400000, 16 * S), E1p, 2, E1p // 4)
    seg_p0 = _mk_segsum(Vp, _rup(V, 16 * S), NG, 1, NG // 2)
    seg_p1 = _mk_segsum(E1p, _rup(E1, 16 * S), NG, 1, NG // 2)
    seg_p2 = _mk_segsum(C2p, _rup(C2, 16 * S), NG, 1, NG // 2)

    s_em, d_em = _pad_edges(v_x, jnp.arange(V, dtype=jnp.int32), _rup(V, 16 * S))
    s_b1, d_b1 = _pad_edges(boundary_index_1[0], boundary_index_1[1],
                            _rup(200000, 16 * S))
    s_b2, d_b2 = _pad_edges(boundary_index_2[0], boundary_index_2[1],
                            _rup(120000, 16 * S))
    s_u0, d_u0 = _pad_edges(up_index_0[0], up_index_0[1], _rup(200000, 16 * S))
    s_u1, d_u1 = _pad_edges(up_index_1[0], up_index_1[1], _rup(400000, 16 * S))
    s_p0, d_p0 = _pad_edges(jnp.arange(V, dtype=jnp.int32), batch0, _rup(V, 16 * S))
    s_p1, d_p1 = _pad_edges(jnp.arange(E1, dtype=jnp.int32), batch1,
                            _rup(E1, 16 * S))
    s_p2, d_p2 = _pad_edges(jnp.arange(C2, dtype=jnp.int32), batch2,
                            _rup(C2, 16 * S))

    x0 = seg_v(params['v_embed'], s_em, d_em)
    x1 = seg_b1(x0, s_b1, d_b1)
    x2 = seg_b2(x1, s_b2, d_b2)

    def pk(p):
        u, bd = p['up'], p['bdry']
        return dict(
            W1=_blockdiag(u['W1'], bd['W1']),
            b1=jnp.concatenate([u['b1'], bd['b1']]).reshape(1, 2 * H),
            g1=jnp.concatenate([u['g1'], bd['g1']]).reshape(1, 2 * H),
            be1=jnp.concatenate([u['be1'], bd['be1']]).reshape(1, 2 * H),
            W2=_blockdiag(u['W2'], bd['W2']),
            b2=jnp.concatenate([u['b2'], bd['b2']]).reshape(1, 2 * H),
            g2=jnp.concatenate([u['g2'], bd['g2']]).reshape(1, 2 * H),
            be2=jnp.concatenate([u['be2'], bd['be2']]).reshape(1, 2 * H),
            Wc=p['comb']['W'], bc=p['comb']['b'].reshape(1, H),
            gc=p['comb']['g'].reshape(1, H), bec=p['comb']['be'].reshape(1, H),
        )

    xs = [x0, x1, x2]
    n_real = [V, E1, C2]
    for l in range(3):
        lp = params['layers'][l]
        aggu = [seg_u0(xs[0], s_u0, d_u0), seg_u1(xs[1], s_u1, d_u1), None]
        aggb = [None, seg_b1(xs[0], s_b1, d_b1), seg_b2(xs[1], s_b2, d_b2)]
        new_xs = []
        for d in range(3):
            p = pk(lp[d])
            n = n_real[d]
            y1, s1 = _pack_mm(xs[d], aggu[d], aggb[d], p['W1'], p['b1'], n)
            y2, s2 = _bn_mm(y1, s1, p['g1'], p['be1'], p['W2'], p['b2'], n)
            y3, s3 = _bn_mm(y2, s2, p['g2'], p['be2'], p['Wc'], p['bc'], n)
            new_xs.append(_bn_act(y3, s3, p['gc'], p['bec'], n))
        xs = new_xs

    pools = [seg_p0(xs[0], s_p0, d_p0), seg_p1(xs[1], s_p1, d_p1),
             seg_p2(xs[2], s_p2, d_p2)]
    ws = [w for w in params['lin1_W']]
    bs = [b.reshape(1, 2 * H) for b in params['lin1_b']]
    return _pool_lin(pools, ws, bs)


# speedup vs baseline: 2.1950x; 2.1945x over previous
import functools

import jax
import jax.numpy as jnp
from jax import lax
from jax.experimental import pallas as pl
from jax.experimental.pallas import tpu as pltpu
from jax.experimental.pallas import tpu_sc as plsc

H = 64
EPS = 1e-5
BN = 1024
G = 128
S = 256
NG = 2048


def _rup(n, m):
    return ((n + m - 1) // m) * m


@functools.lru_cache(maxsize=None)
def _mk_segsum(n_x, e_pad, m_pad, nblk, br):
    assert e_pad % (16 * S) == 0 and m_pad == 2 * nblk * br and br % 16 == 0
    et = e_pad // 16
    nslab = et // S
    nsub = S // G
    brt = (br + G) // 16
    brs = br // 16
    mesh = plsc.VectorSubcoreMesh(core_axis_name="c", subcore_axis_name="s")

    @functools.partial(
        pl.kernel, mesh=mesh,
        compiler_params=pltpu.CompilerParams(use_tc_tiling_on_sc=False),
        out_type=jax.ShapeDtypeStruct((m_pad, H), jnp.float32),
        scratch_types=[
            pltpu.VMEM((S,), jnp.int32),
            pltpu.VMEM((S,), jnp.int32),
            pltpu.VMEM((G,), jnp.int32),
            pltpu.VMEM((G,), jnp.int32),
            pltpu.VMEM((G,), jnp.int32),
            pltpu.VMEM((G,), jnp.int32),
            pltpu.VMEM((G, H), jnp.float32),
            pltpu.VMEM((G, H), jnp.float32),
            pltpu.VMEM((G, H), jnp.float32),
            pltpu.VMEM_SHARED((br + G, H), jnp.float32),
            pltpu.SemaphoreType.DMA,
            pltpu.SemaphoreType.DMA,
        ],
    )
    def seg(x_hbm, src_hbm, dst_hbm, out_hbm,
            dslab, sslab, dstm0, dstm1, gidx0, gidx1, rows0, rows1,
            zbuf, acc, sem0, sem1):
        cid = lax.axis_index("c")
        sid = lax.axis_index("s")
        dstm = [dstm0, dstm1]
        gidx = [gidx0, gidx1]
        rows = [rows0, rows1]
        sem = [sem0, sem1]
        zv = jnp.zeros((16,), jnp.float32)

        def zrow(r, _):
            for v in range(H // 16):
                zbuf[r, pl.ds(v * 16, 16)] = zv
            return 0
        lax.fori_loop(0, G, zrow, 0)

        e0 = sid * et
        for blk in range(nblk):
            lo = (cid * nblk + blk) * br
            nz = brt // G
            rem = brt - nz * G

            def zb(z, _):
                pltpu.sync_copy(zbuf, acc.at[pl.ds(sid * brt + z * G, G)])
                return 0
            lax.fori_loop(0, nz, zb, 0)
            if rem:
                pltpu.sync_copy(zbuf.at[pl.ds(0, rem)],
                                acc.at[pl.ds(sid * brt + nz * G, rem)])
            plsc.subcore_barrier()

            def slab_body(t, _):
                eoff = e0 + t * S
                pltpu.sync_copy(dst_hbm.at[pl.ds(eoff, S)], dslab)
                pltpu.sync_copy(src_hbm.at[pl.ds(eoff, S)], sslab)
                cps = []
                for k in range(nsub):
                    p = k % 2
                    for v in range(G // 16):
                        dv = dslab[pl.ds(k * G + v * 16, 16)]
                        inr = (dv >= lo) & (dv < lo + br)
                        dstm[p][pl.ds(v * 16, 16)] = jnp.where(inr, dv - lo, br)
                        gidx[p][pl.ds(v * 16, 16)] = sslab[pl.ds(k * G + v * 16, 16)]
                    cps.append(pltpu.async_copy(
                        x_hbm.at[gidx[p]], rows[p], sem[p]))
                    if k > 0:
                        cps[k - 1].wait()
                        pltpu.sync_copy(rows[1 - p], acc.at[dstm[1 - p]],
                                        add=True)
                cps[nsub - 1].wait()
                q = (nsub - 1) % 2
                pltpu.sync_copy(rows[q], acc.at[dstm[q]], add=True)
                return 0
            lax.fori_loop(0, nslab, slab_body, 0)
            plsc.subcore_barrier()
            pltpu.sync_copy(acc.at[pl.ds(sid * brs, brs)],
                            out_hbm.at[pl.ds(lo + sid * brs, brs)])
            plsc.subcore_barrier()

    return seg


def _stats_update(i, y, s_ref, nreal):
    rows = pl.program_id(0) * BN + lax.broadcasted_iota(jnp.int32, (BN, 1), 0)
    ym = jnp.where(rows < nreal, y, 0.0)
    s = jnp.stack([ym.sum(0), (ym * ym).sum(0)], axis=0)

    @pl.when(i == 0)
    def _():
        s_ref[...] = s

    @pl.when(i > 0)
    def _():
        s_ref[...] += s


def _pack_mm(x, aggu, aggb, w, b, nreal):
    np_, _ = x.shape
    grid = np_ // BN
    has_u, has_b = aggu is not None, aggb is not None

    def body(*refs):
        i = pl.program_id(0)
        x_ = refs[0][...]
        k = 1
        if has_u:
            xu = x_ + refs[k][...]
            k += 1
        else:
            xu = x_
        if has_b:
            xb = x_ + refs[k][...]
            k += 1
        else:
            xb = x_
        w_, b_ = refs[k][...], refs[k + 1][...]
        y_ref, s_ref = refs[k + 2], refs[k + 3]
        cat = jnp.concatenate([xu, xb], axis=1)
        y = jnp.dot(cat, w_, preferred_element_type=jnp.float32) + b_
        y_ref[...] = y
        _stats_update(i, y, s_ref, nreal)

    ins = [x] + ([aggu] if has_u else []) + ([aggb] if has_b else []) + [w, b]
    in_specs = ([pl.BlockSpec((BN, H), lambda i: (i, 0))] * (1 + has_u + has_b)
                + [pl.BlockSpec((2 * H, 2 * H), lambda i: (0, 0)),
                   pl.BlockSpec((1, 2 * H), lambda i: (0, 0))])
    return pl.pallas_call(
        body, grid=(grid,),
        in_specs=in_specs,
        out_specs=[pl.BlockSpec((BN, 2 * H), lambda i: (i, 0)),
                   pl.BlockSpec((2, 2 * H), lambda i: (0, 0))],
        out_shape=[jax.ShapeDtypeStruct((np_, 2 * H), jnp.float32),
                   jax.ShapeDtypeStruct((2, 2 * H), jnp.float32)],
    )(*ins)


def _bn_scale_shift(s, g, be, nreal):
    mean = s[0:1] / nreal
    var = s[1:2] / nreal - mean * mean
    inv = g * lax.rsqrt(var + EPS)
    return inv, be - mean * inv


def _bn_mm(y_in, s_in, g, be, w, b, nreal):
    np_, hin = y_in.shape
    hout = w.shape[1]
    grid = np_ // BN

    def body(y_ref, s_in_ref, g_ref, be_ref, w_ref, b_ref, o_ref, s_ref):
        i = pl.program_id(0)
        inv, sh = _bn_scale_shift(s_in_ref[...], g_ref[...], be_ref[...], nreal)
        h = jnp.maximum(y_ref[...] * inv + sh, 0.0)
        o = jnp.dot(h, w_ref[...], preferred_element_type=jnp.float32) + b_ref[...]
        o_ref[...] = o
        _stats_update(i, o, s_ref, nreal)

    return pl.pallas_call(
        body, grid=(grid,),
        in_specs=[pl.BlockSpec((BN, hin), lambda i: (i, 0)),
                  pl.BlockSpec((2, hin), lambda i: (0, 0)),
                  pl.BlockSpec((1, hin), lambda i: (0, 0)),
                  pl.BlockSpec((1, hin), lambda i: (0, 0)),
                  pl.BlockSpec((hin, hout), lambda i: (0, 0)),
                  pl.BlockSpec((1, hout), lambda i: (0, 0))],
        out_specs=[pl.BlockSpec((BN, hout), lambda i: (i, 0)),
                   pl.BlockSpec((2, hout), lambda i: (0, 0))],
        out_shape=[jax.ShapeDtypeStruct((np_, hout), jnp.float32),
                   jax.ShapeDtypeStruct((2, hout), jnp.float32)],
    )(y_in, s_in, g, be, w, b)


def _bn_act(y_in, s_in, g, be, nreal):
    np_, hin = y_in.shape
    grid = np_ // BN

    def body(y_ref, s_in_ref, g_ref, be_ref, o_ref):
        inv, sh = _bn_scale_shift(s_in_ref[...], g_ref[...], be_ref[...], nreal)
        o_ref[...] = jnp.maximum(y_ref[...] * inv + sh, 0.0)

    return pl.pallas_call(
        body, grid=(grid,),
        in_specs=[pl.BlockSpec((BN, hin), lambda i: (i, 0)),
                  pl.BlockSpec((2, hin), lambda i: (0, 0)),
                  pl.BlockSpec((1, hin), lambda i: (0, 0)),
                  pl.BlockSpec((1, hin), lambda i: (0, 0))],
        out_specs=pl.BlockSpec((BN, hin), lambda i: (i, 0)),
        out_shape=jax.ShapeDtypeStruct((np_, hin), jnp.float32),
    )(y_in, s_in, g, be)


def _pool_lin(pools, ws, bs):
    def body(p0, p1, p2, w0, w1, w2, b0, b1, b2, o_ref):
        o = jnp.maximum(jnp.dot(p0[...], w0[...],
                                preferred_element_type=jnp.float32) + b0[...], 0.0)
        o += jnp.maximum(jnp.dot(p1[...], w1[...],
                                 preferred_element_type=jnp.float32) + b1[...], 0.0)
        o += jnp.maximum(jnp.dot(p2[...], w2[...],
                                 preferred_element_type=jnp.float32) + b2[...], 0.0)
        o_ref[...] = o

    specs = ([pl.BlockSpec((NG, H), lambda: (0, 0))] * 3
             + [pl.BlockSpec((H, 2 * H), lambda: (0, 0))] * 3
             + [pl.BlockSpec((1, 2 * H), lambda: (0, 0))] * 3)
    return pl.pallas_call(
        body,
        in_specs=specs,
        out_specs=pl.BlockSpec((NG, 2 * H), lambda: (0, 0)),
        out_shape=jax.ShapeDtypeStruct((NG, 2 * H), jnp.float32),
    )(*pools, *ws, *bs)


def _blockdiag(a, b):
    z = jnp.zeros_like(a)
    return jnp.concatenate(
        [jnp.concatenate([a, z], 1), jnp.concatenate([z, b], 1)], 0)


def _pad_edges(src, dst, e_pad):
    e = src.shape[0]
    ps = jnp.concatenate([src.astype(jnp.int32),
                          jnp.zeros((e_pad - e,), jnp.int32)])
    pd = jnp.concatenate([dst.astype(jnp.int32),
                          jnp.full((e_pad - e,), -1, jnp.int32)])
    return ps, pd


def kernel(params, v_x, up_index_0, boundary_index_1, up_index_1,
           boundary_index_2, batch0, batch1, batch2):
    V, E1, C2 = v_x.shape[0], batch1.shape[0], batch2.shape[0]
    Vp, E1p, C2p = _rup(V, BN), _rup(E1, BN), _rup(C2, BN)

    seg_v = _mk_segsum(params['v_embed'].shape[0], _rup(V, 16 * S), Vp, 1, Vp // 2)
    seg_b1 = _mk_segsum(Vp, _rup(200000, 16 * S), E1p, 2, E1p // 4)
    seg_b2 = _mk_segsum(E1p, _rup(120000, 16 * S), C2p, 1, C2p // 2)
    seg_u0 = _mk_segsum(Vp, _rup(200000, 16 * S), Vp, 1, Vp // 2)
    seg_u1 = _mk_segsum(E1p, _rup(400000, 16 * S), E1p, 2, E1p // 4)
    seg_p0 = _mk_segsum(Vp, _rup(V, 16 * S), NG, 1, NG // 2)
    seg_p1 = _mk_segsum(E1p, _rup(E1, 16 * S), NG, 1, NG // 2)
    seg_p2 = _mk_segsum(C2p, _rup(C2, 16 * S), NG, 1, NG // 2)

    s_em, d_em = _pad_edges(v_x, jnp.arange(V, dtype=jnp.int32), _rup(V, 16 * S))
    s_b1, d_b1 = _pad_edges(boundary_index_1[0], boundary_index_1[1],
                            _rup(200000, 16 * S))
    s_b2, d_b2 = _pad_edges(boundary_index_2[0], boundary_index_2[1],
                            _rup(120000, 16 * S))
    s_u0, d_u0 = _pad_edges(up_index_0[0], up_index_0[1], _rup(200000, 16 * S))
    s_u1, d_u1 = _pad_edges(up_index_1[0], up_index_1[1], _rup(400000, 16 * S))
    s_p0, d_p0 = _pad_edges(jnp.arange(V, dtype=jnp.int32), batch0, _rup(V, 16 * S))
    s_p1, d_p1 = _pad_edges(jnp.arange(E1, dtype=jnp.int32), batch1,
                            _rup(E1, 16 * S))
    s_p2, d_p2 = _pad_edges(jnp.arange(C2, dtype=jnp.int32), batch2,
                            _rup(C2, 16 * S))

    x0 = seg_v(params['v_embed'], s_em, d_em)
    x1 = seg_b1(x0, s_b1, d_b1)
    x2 = seg_b2(x1, s_b2, d_b2)

    def pk(p):
        u, bd = p['up'], p['bdry']
        return dict(
            W1=_blockdiag(u['W1'], bd['W1']),
            b1=jnp.concatenate([u['b1'], bd['b1']]).reshape(1, 2 * H),
            g1=jnp.concatenate([u['g1'], bd['g1']]).reshape(1, 2 * H),
            be1=jnp.concatenate([u['be1'], bd['be1']]).reshape(1, 2 * H),
            W2=_blockdiag(u['W2'], bd['W2']),
            b2=jnp.concatenate([u['b2'], bd['b2']]).reshape(1, 2 * H),
            g2=jnp.concatenate([u['g2'], bd['g2']]).reshape(1, 2 * H),
            be2=jnp.concatenate([u['be2'], bd['be2']]).reshape(1, 2 * H),
            Wc=p['comb']['W'], bc=p['comb']['b'].reshape(1, H),
            gc=p['comb']['g'].reshape(1, H), bec=p['comb']['be'].reshape(1, H),
        )

    xs = [x0, x1, x2]
    n_real = [V, E1, C2]
    for l in range(3):
        lp = params['layers'][l]
        aggu = [seg_u0(xs[0], s_u0, d_u0), seg_u1(xs[1], s_u1, d_u1), None]
        aggb = [None, seg_b1(xs[0], s_b1, d_b1), seg_b2(xs[1], s_b2, d_b2)]
        new_xs = []
        for d in range(3):
            p = pk(lp[d])
            n = n_real[d]
            y1, s1 = _pack_mm(xs[d], aggu[d], aggb[d], p['W1'], p['b1'], n)
            y2, s2 = _bn_mm(y1, s1, p['g1'], p['be1'], p['W2'], p['b2'], n)
            y3, s3 = _bn_mm(y2, s2, p['g2'], p['be2'], p['Wc'], p['bc'], n)
            new_xs.append(_bn_act(y3, s3, p['gc'], p['bec'], n))
        xs = new_xs

    pools = [seg_p0(xs[0], s_p0, d_p0), seg_p1(xs[1], s_p1, d_p1),
             seg_p2(xs[2], s_p2, d_p2)]
    ws = [w for w in params['lin1_W']]
    bs = [b.reshape(1, 2 * H) for b in params['lin1_b']]
    return _pool_lin(pools, ws, bs)
